# Initial kernel scaffold; baseline (speedup 1.0000x reference)
import jax
import jax.numpy as jnp
from jax.experimental import pallas as pl


def kernel(v_logits, t_logits, init_weights, W1, b1, g1, be1, W2, b2, W3, b3, g3, be3, W4, b4, g4, be4, W5, b5, bin_center):
    raise NotImplementedError("write your pallas kernel here")


# speedup vs baseline: 7.4401x; 7.4401x over previous
import jax
import jax.numpy as jnp
from jax.experimental import pallas as pl
from jax.experimental.pallas import tpu as pltpu

B = 16384
C = 102
R = 2048
N = B // R
H = 32


def _fusion_kernel(v_ref, t_ref, iw_ref, W1_ref, b1_ref, g1_ref, be1_ref,
                   W2_ref, b2_ref, W3_ref, b3_ref, g3_ref, be3_ref,
                   W4_ref, b4_ref, g4_ref, be4_ref, W5_ref, b5_ref, bc_ref,
                   out_ref, h1v_ref, h1t_ref, w_ref):
    i = pl.program_id(0)

    @pl.when(i < N)
    def _embed_phase():
        iw = iw_ref[...]
        cols = jax.lax.broadcasted_iota(jnp.int32, (R, C), 1)

        def h1_of(l):
            m = jnp.max(l, axis=1, keepdims=True)
            ex = jnp.exp(l - m)
            z = jnp.sum(ex, axis=1, keepdims=True)
            p = ex / z
            lp = jnp.log(p + 1e-8)
            ent = -jnp.sum(p * lp, axis=1, keepdims=True)
            conf = jnp.max(p, axis=1, keepdims=True)
            feats = [ent, conf]
            pk = p
            for _ in range(3):
                mk = jnp.max(pk, axis=1, keepdims=True)
                ik = jnp.min(jnp.where(pk == mk, cols, C), axis=1,
                             keepdims=True)
                sel = cols == ik
                feats.append(jnp.sum(jnp.where(sel, iw, 0.0), axis=1,
                                     keepdims=True))
                pk = jnp.where(sel, -1.0, pk)
            acc = jnp.broadcast_to(b1_ref[...], (R, H))
            for k, f in enumerate(feats):
                acc = acc + f * W1_ref[k:k + 1, :]
            return acc

        h1v_ref[pl.ds(i * R, R), :] = h1_of(v_ref[...])
        h1t_ref[pl.ds(i * R, R), :] = h1_of(t_ref[...])

    @pl.when(i == N - 1)
    def _mlp_phase():
        def bn(x, g, b):
            mu = jnp.mean(x, axis=0, keepdims=True)
            var = jnp.mean((x - mu) ** 2, axis=0, keepdims=True)
            return g * (x - mu) / jnp.sqrt(var + 1e-5) + b

        def proj(h):
            h = jnp.maximum(bn(h, g1_ref[...], be1_ref[...]), 0.0)
            return (jnp.dot(h, W2_ref[...],
                            preferred_element_type=jnp.float32)
                    + b2_ref[...])

        vf = proj(h1v_ref[...])
        tf = proj(h1t_ref[...])
        h = (jnp.dot(vf, W3_ref[0:H, :], preferred_element_type=jnp.float32)
             + jnp.dot(tf, W3_ref[H:2 * H, :],
                       preferred_element_type=jnp.float32)
             + b3_ref[...])
        h = jnp.maximum(bn(h, g3_ref[...], be3_ref[...]), 0.0)
        h = jnp.dot(h, W4_ref[...], preferred_element_type=jnp.float32) \
            + b4_ref[...]
        h = jnp.maximum(bn(h, g4_ref[...], be4_ref[...]), 0.0)
        gate = jnp.dot(h, W5_ref[...], preferred_element_type=jnp.float32) \
            + b5_ref[...]
        gm = jnp.max(gate, axis=1, keepdims=True)
        ge = jnp.exp(gate - gm)
        gp = ge / jnp.sum(ge, axis=1, keepdims=True)
        w_ref[...] = jnp.sum(gp * bc_ref[...], axis=1, keepdims=True)

    @pl.when(i >= N)
    def _fuse_phase():
        w = w_ref[pl.ds((i - N) * R, R), :]
        out_ref[...] = w * v_ref[...] + (2.0 - w) * t_ref[...]


def kernel(v_logits, t_logits, init_weights, W1, b1, g1, be1, W2, b2,
           W3, b3, g3, be3, W4, b4, g4, be4, W5, b5, bin_center):
    row2d = lambda a: a.reshape(1, -1)
    logits_map = lambda i: (jnp.where(i < N, i, i - N), 0)
    fixed = lambda shape: pl.BlockSpec(shape, lambda i: (0, 0))

    return pl.pallas_call(
        _fusion_kernel,
        grid=(2 * N,),
        in_specs=[
            pl.BlockSpec((R, C), logits_map),
            pl.BlockSpec((R, C), logits_map),
            fixed((1, C)),
            fixed((5, H)),
            fixed((1, H)),
            fixed((1, H)),
            fixed((1, H)),
            fixed((H, H)),
            fixed((1, H)),
            fixed((2 * H, H)),
            fixed((1, H)),
            fixed((1, H)),
            fixed((1, H)),
            fixed((H, H)),
            fixed((1, H)),
            fixed((1, H)),
            fixed((1, H)),
            fixed((H, 9)),
            fixed((1, 9)),
            fixed((1, 9)),
        ],
        out_specs=pl.BlockSpec((R, C),
                               lambda i: (jnp.where(i < N, 0, i - N), 0)),
        out_shape=jax.ShapeDtypeStruct((B, C), jnp.float32),
        scratch_shapes=[
            pltpu.VMEM((B, H), jnp.float32),
            pltpu.VMEM((B, H), jnp.float32),
            pltpu.VMEM((B, 1), jnp.float32),
        ],
    )(v_logits, t_logits, row2d(init_weights), W1, row2d(b1), row2d(g1),
      row2d(be1), W2, row2d(b2), W3, row2d(b3), row2d(g3), row2d(be3),
      W4, row2d(b4), row2d(g4), row2d(be4), W5, row2d(b5), row2d(bin_center))


# speedup vs baseline: 7.4493x; 1.0012x over previous
import jax
import jax.numpy as jnp
from jax.experimental import pallas as pl
from jax.experimental.pallas import tpu as pltpu

B = 16384
C = 102
R = 2048
N = B // R
H = 32


def _fusion_kernel(v_ref, t_ref, iw_ref, W1_ref, b1_ref, g1_ref, be1_ref,
                   W2_ref, b2_ref, W3_ref, b3_ref, g3_ref, be3_ref,
                   W4_ref, b4_ref, g4_ref, be4_ref, W5_ref, b5_ref, bc_ref,
                   out_ref, vs_ref, ts_ref, h1_ref):
    i = pl.program_id(0)

    @pl.when(i < N)
    def _embed_phase():
        iw = iw_ref[...]
        cols = jax.lax.broadcasted_iota(jnp.int32, (R, C), 1)
        v = v_ref[...]
        t = t_ref[...]
        vs_ref[pl.ds(i * R, R), :] = v
        ts_ref[pl.ds(i * R, R), :] = t

        def h1_of(l):
            m = jnp.max(l, axis=1, keepdims=True)
            x = l - m
            ex = jnp.exp(x)
            z = jnp.sum(ex, axis=1, keepdims=True)
            rz = 1.0 / z
            conf = rz
            ent = jnp.log(z) - jnp.sum(ex * x, axis=1, keepdims=True) * rz
            feats = [ent, conf]
            pk = ex * rz
            for _ in range(3):
                mk = jnp.max(pk, axis=1, keepdims=True)
                ik = jnp.min(jnp.where(pk == mk, cols, C), axis=1,
                             keepdims=True)
                sel = cols == ik
                feats.append(jnp.sum(jnp.where(sel, iw, 0.0), axis=1,
                                     keepdims=True))
                pk = jnp.where(sel, -1.0, pk)
            acc = jnp.broadcast_to(b1_ref[...], (R, H))
            for k, f in enumerate(feats):
                acc = acc + f * W1_ref[k:k + 1, :]
            return acc

        h1_ref[pl.ds(i * R, R), 0:H] = h1_of(v)
        h1_ref[pl.ds(i * R, R), H:2 * H] = h1_of(t)

    @pl.when(i == N - 1)
    def _mlp_phase():
        def bn(x, g, b):
            mu = jnp.mean(x, axis=0, keepdims=True)
            var = jnp.mean((x - mu) ** 2, axis=0, keepdims=True)
            return g * (x - mu) / jnp.sqrt(var + 1e-5) + b

        def proj(h):
            h = jnp.maximum(bn(h, g1_ref[...], be1_ref[...]), 0.0)
            return (jnp.dot(h, W2_ref[...],
                            preferred_element_type=jnp.float32)
                    + b2_ref[...])

        vf = proj(h1_ref[:, 0:H])
        tf = proj(h1_ref[:, H:2 * H])
        h = (jnp.dot(vf, W3_ref[0:H, :], preferred_element_type=jnp.float32)
             + jnp.dot(tf, W3_ref[H:2 * H, :],
                       preferred_element_type=jnp.float32)
             + b3_ref[...])
        h = jnp.maximum(bn(h, g3_ref[...], be3_ref[...]), 0.0)
        h = jnp.dot(h, W4_ref[...], preferred_element_type=jnp.float32) \
            + b4_ref[...]
        h = jnp.maximum(bn(h, g4_ref[...], be4_ref[...]), 0.0)
        gate = jnp.dot(h, W5_ref[...], preferred_element_type=jnp.float32) \
            + b5_ref[...]
        gm = jnp.max(gate, axis=1, keepdims=True)
        ge = jnp.exp(gate - gm)
        gp = ge / jnp.sum(ge, axis=1, keepdims=True)
        h1_ref[:, 2 * H:2 * H + 1] = jnp.sum(gp * bc_ref[...], axis=1,
                                             keepdims=True)

    @pl.when(i >= N)
    def _fuse_phase():
        j = i - N
        w = h1_ref[pl.ds(j * R, R), 2 * H:2 * H + 1]
        out_ref[...] = (w * vs_ref[pl.ds(j * R, R), :]
                        + (2.0 - w) * ts_ref[pl.ds(j * R, R), :])


def kernel(v_logits, t_logits, init_weights, W1, b1, g1, be1, W2, b2,
           W3, b3, g3, be3, W4, b4, g4, be4, W5, b5, bin_center):
    row2d = lambda a: a.reshape(1, -1)
    logits_map = lambda i: (jnp.minimum(i, N - 1), 0)
    fixed = lambda shape: pl.BlockSpec(shape, lambda i: (0, 0))

    return pl.pallas_call(
        _fusion_kernel,
        grid=(2 * N,),
        in_specs=[
            pl.BlockSpec((R, C), logits_map),
            pl.BlockSpec((R, C), logits_map),
            fixed((1, C)),
            fixed((5, H)),
            fixed((1, H)),
            fixed((1, H)),
            fixed((1, H)),
            fixed((H, H)),
            fixed((1, H)),
            fixed((2 * H, H)),
            fixed((1, H)),
            fixed((1, H)),
            fixed((1, H)),
            fixed((H, H)),
            fixed((1, H)),
            fixed((1, H)),
            fixed((1, H)),
            fixed((H, 9)),
            fixed((1, 9)),
            fixed((1, 9)),
        ],
        out_specs=pl.BlockSpec((R, C),
                               lambda i: (jnp.where(i < N, 0, i - N), 0)),
        out_shape=jax.ShapeDtypeStruct((B, C), jnp.float32),
        scratch_shapes=[
            pltpu.VMEM((B, C), jnp.float32),
            pltpu.VMEM((B, C), jnp.float32),
            pltpu.VMEM((B, 2 * H + 1), jnp.float32),
        ],
    )(v_logits, t_logits, row2d(init_weights), W1, row2d(b1), row2d(g1),
      row2d(be1), W2, row2d(b2), W3, row2d(b3), row2d(g3), row2d(be3),
      W4, row2d(b4), row2d(g4), row2d(be4), W5, row2d(b5), row2d(bin_center))


# speedup vs baseline: 9.6229x; 1.2918x over previous
import jax
import jax.numpy as jnp
from jax.experimental import pallas as pl
from jax.experimental.pallas import tpu as pltpu

B = 16384
C = 102
R = 2048
N = B // R
H = 32


def _fusion_kernel(v_ref, t_ref, iw_ref, W1_ref, b1_ref, g1_ref, be1_ref,
                   W2_ref, b2_ref, W3_ref, b3_ref, g3_ref, be3_ref,
                   W4_ref, b4_ref, g4_ref, be4_ref, W5_ref, b5_ref, bc_ref,
                   out_ref, vs_ref, ts_ref, e_ref):
    i = pl.program_id(0)

    @pl.when(i < N)
    def _embed_phase():
        iw = iw_ref[...]
        cols = jax.lax.broadcasted_iota(jnp.int32, (R, C), 1)
        v = v_ref[...]
        t = t_ref[...]
        vs_ref[pl.ds(i * R, R), :] = v
        ts_ref[pl.ds(i * R, R), :] = t

        def feats_of(l):
            m = jnp.max(l, axis=1, keepdims=True)
            x = l - m
            ex = jnp.exp(x)
            z = jnp.sum(ex, axis=1, keepdims=True)
            rz = 1.0 / z
            conf = rz
            ent = jnp.log(z) - jnp.sum(ex * x, axis=1, keepdims=True) * rz
            feats = [ent, conf]
            pk = ex * rz
            for _ in range(3):
                mk = jnp.max(pk, axis=1, keepdims=True)
                ik = jnp.min(jnp.where(pk == mk, cols, C), axis=1,
                             keepdims=True)
                sel = cols == ik
                feats.append(jnp.sum(jnp.where(sel, iw, 0.0), axis=1,
                                     keepdims=True))
                pk = jnp.where(sel, -1.0, pk)
            return jnp.concatenate(feats, axis=1)

        e_ref[pl.ds(i * R, R), 0:5] = feats_of(v)
        e_ref[pl.ds(i * R, R), 5:10] = feats_of(t)

    @pl.when(i == N - 1)
    def _mlp_phase():
        rb = 1.0 / B

        def bn_relu(x, g, b):
            mu = jnp.sum(x, axis=0, keepdims=True) * rb
            m2 = jnp.sum(x * x, axis=0, keepdims=True) * rb
            a = g * jax.lax.rsqrt(m2 - mu * mu + 1e-5)
            return jnp.maximum(x * a + (b - mu * a), 0.0)

        two = lambda r: jnp.concatenate([r, r], axis=1)
        W1 = W1_ref[...]
        z5 = jnp.zeros((5, H), jnp.float32)
        Wbig = jnp.concatenate(
            [jnp.concatenate([W1, z5], axis=1),
             jnp.concatenate([z5, W1], axis=1)], axis=0)
        e = e_ref[:, 0:10]
        h1 = jnp.dot(e, Wbig, preferred_element_type=jnp.float32) \
            + two(b1_ref[...])
        x1 = bn_relu(h1, two(g1_ref[...]), two(be1_ref[...]))

        W3a = W3_ref[0:H, :]
        W3b = W3_ref[H:2 * H, :]
        Wc = jnp.concatenate(
            [jnp.dot(W2_ref[...], W3a, preferred_element_type=jnp.float32),
             jnp.dot(W2_ref[...], W3b, preferred_element_type=jnp.float32)],
            axis=0)
        bc3 = jnp.dot(b2_ref[...], W3a + W3b,
                      preferred_element_type=jnp.float32) + b3_ref[...]
        h3 = jnp.dot(x1, Wc, preferred_element_type=jnp.float32) + bc3
        x3 = bn_relu(h3, g3_ref[...], be3_ref[...])
        h4 = jnp.dot(x3, W4_ref[...], preferred_element_type=jnp.float32) \
            + b4_ref[...]
        x4 = bn_relu(h4, g4_ref[...], be4_ref[...])
        gate = jnp.dot(x4, W5_ref[...], preferred_element_type=jnp.float32) \
            + b5_ref[...]
        gm = jnp.max(gate, axis=1, keepdims=True)
        ge = jnp.exp(gate - gm)
        num = jnp.sum(ge * bc_ref[...], axis=1, keepdims=True)
        den = jnp.sum(ge, axis=1, keepdims=True)
        e_ref[:, 10:11] = num / den

    @pl.when(i >= N)
    def _fuse_phase():
        j = i - N
        w = e_ref[pl.ds(j * R, R), 10:11]
        out_ref[...] = (w * vs_ref[pl.ds(j * R, R), :]
                        + (2.0 - w) * ts_ref[pl.ds(j * R, R), :])


def kernel(v_logits, t_logits, init_weights, W1, b1, g1, be1, W2, b2,
           W3, b3, g3, be3, W4, b4, g4, be4, W5, b5, bin_center):
    row2d = lambda a: a.reshape(1, -1)
    logits_map = lambda i: (jnp.minimum(i, N - 1), 0)
    fixed = lambda shape: pl.BlockSpec(shape, lambda i: (0, 0))

    return pl.pallas_call(
        _fusion_kernel,
        grid=(2 * N,),
        in_specs=[
            pl.BlockSpec((R, C), logits_map),
            pl.BlockSpec((R, C), logits_map),
            fixed((1, C)),
            fixed((5, H)),
            fixed((1, H)),
            fixed((1, H)),
            fixed((1, H)),
            fixed((H, H)),
            fixed((1, H)),
            fixed((2 * H, H)),
            fixed((1, H)),
            fixed((1, H)),
            fixed((1, H)),
            fixed((H, H)),
            fixed((1, H)),
            fixed((1, H)),
            fixed((1, H)),
            fixed((H, 9)),
            fixed((1, 9)),
            fixed((1, 9)),
        ],
        out_specs=pl.BlockSpec((R, C),
                               lambda i: (jnp.where(i < N, 0, i - N), 0)),
        out_shape=jax.ShapeDtypeStruct((B, C), jnp.float32),
        scratch_shapes=[
            pltpu.VMEM((B, C), jnp.float32),
            pltpu.VMEM((B, C), jnp.float32),
            pltpu.VMEM((B, 16), jnp.float32),
        ],
    )(v_logits, t_logits, row2d(init_weights), W1, row2d(b1), row2d(g1),
      row2d(be1), W2, row2d(b2), W3, row2d(b3), row2d(g3), row2d(be3),
      W4, row2d(b4), row2d(g4), row2d(be4), W5, row2d(b5), row2d(bin_center))
